# baseline (speedup 1.0000x reference)
import functools

import jax
import jax.numpy as jnp
from jax import lax
from jax.experimental import pallas as pl
from jax.experimental.pallas import tpu as pltpu
from jax.experimental.pallas import tpu_sc as plsc

_L = 16


@jax.jit
def _sc_gather(flat_rows, positions):
    num_rows, d = flat_rows.shape
    mb, t = positions.shape
    seq_len = num_rows // mb
    info = plsc.get_sparse_core_info()
    nc, ns = info.num_cores, info.num_subcores
    nw = nc * ns
    w_pb = nw // mb
    b_per_w = t // w_pb
    n_full = b_per_w // _L
    tail = b_per_w - n_full * _L

    mesh = plsc.VectorSubcoreMesh(core_axis_name="c", subcore_axis_name="s")

    @functools.partial(
        pl.kernel,
        mesh=mesh,
        out_type=jax.ShapeDtypeStruct((mb, t, d), jnp.float32),
        scratch_types=[
            pltpu.VMEM((b_per_w,), jnp.int32),
            pltpu.VMEM((b_per_w, d), jnp.float32),
            pltpu.SemaphoreType.DMA,
        ],
        compiler_params=pltpu.CompilerParams(use_tc_tiling_on_sc=False),
    )
    def k(table_hbm, pos_hbm, out_hbm, idx_v, rows_v, sem):
        wid = lax.axis_index("s") * nc + lax.axis_index("c")
        batch = wid // w_pb
        col0 = (wid % w_pb) * b_per_w
        offset = batch * jnp.int32(seq_len)
        pltpu.sync_copy(pos_hbm.at[batch, pl.ds(col0, b_per_w)], idx_v)
        for i in range(n_full):
            sl = pl.ds(i * _L, _L)
            idx_v[sl] = idx_v[sl] + offset
        if tail:
            sl = pl.ds(b_per_w - _L, _L)
            lane = lax.iota(jnp.int32, _L)
            idx_v[sl] = idx_v[sl] + jnp.where(lane >= _L - tail, offset, 0)
        pltpu.async_copy(table_hbm.at[idx_v], rows_v, sem).wait()
        pltpu.sync_copy(rows_v, out_hbm.at[batch, pl.ds(col0, b_per_w)])

    return k(flat_rows, positions)


def kernel(inputs, positions):
    mb, s, d = inputs.shape
    return _sc_gather(inputs.reshape(mb * s, d), positions)


# speedup vs baseline: 2.4581x; 2.4581x over previous
import functools

import jax
import jax.numpy as jnp
from jax import lax
from jax.experimental import pallas as pl
from jax.experimental.pallas import tpu as pltpu
from jax.experimental.pallas import tpu_sc as plsc

_L = 16


@functools.partial(jax.jit, static_argnames=("seq_len", "tok_per_batch"))
def _sc_gather(flat_rows, idx_flat, seq_len, tok_per_batch):
    num_rows, d = flat_rows.shape
    n_idx = idx_flat.shape[0]
    info = plsc.get_sparse_core_info()
    nc, ns = info.num_cores, info.num_subcores
    nw = nc * ns
    b_per_w = n_idx // nw
    n_full = b_per_w // _L
    tail = b_per_w - n_full * _L

    mesh = plsc.VectorSubcoreMesh(core_axis_name="c", subcore_axis_name="s")

    @functools.partial(
        pl.kernel,
        mesh=mesh,
        out_type=jax.ShapeDtypeStruct((n_idx, d), jnp.float32),
        scratch_types=[
            pltpu.VMEM((b_per_w,), jnp.int32),
            pltpu.VMEM((b_per_w, d), jnp.float32),
            pltpu.SemaphoreType.DMA,
        ],
    )
    def k(table_hbm, idx_hbm, out_hbm, idx_v, rows_v, sem):
        wid = lax.axis_index("s") * nc + lax.axis_index("c")
        base = wid * b_per_w
        offset = (base // tok_per_batch) * jnp.int32(seq_len)
        pltpu.sync_copy(idx_hbm.at[pl.ds(base, b_per_w)], idx_v)
        for i in range(n_full):
            sl = pl.ds(i * _L, _L)
            idx_v[sl] = idx_v[sl] + offset
        if tail:
            sl = pl.ds(b_per_w - _L, _L)
            lane = lax.iota(jnp.int32, _L)
            idx_v[sl] = idx_v[sl] + jnp.where(lane >= _L - tail, offset, 0)
        pltpu.async_copy(table_hbm.at[idx_v], rows_v, sem).wait()
        pltpu.sync_copy(rows_v, out_hbm.at[pl.ds(base, b_per_w)])

    return k(flat_rows, idx_flat)


def kernel(inputs, positions):
    mb, s, d = inputs.shape
    _, t = positions.shape
    flat_rows = inputs.reshape(mb * s, d)
    pos_flat = positions.reshape(mb * t)
    out = _sc_gather(flat_rows, pos_flat, s, t)
    return out.reshape(mb, t, d)
